# Initial kernel scaffold; baseline (speedup 1.0000x reference)
import jax
import jax.numpy as jnp
from jax.experimental import pallas as pl


def kernel(world_pos, prev_world_pos, node_type, mesh_pos, cells, params):
    raise NotImplementedError("write your pallas kernel here")


# speedup vs baseline: 2.1652x; 2.1652x over previous
import functools

import jax
import jax.numpy as jnp
from jax import lax
from jax.experimental import pallas as pl
from jax.experimental.pallas import tpu as pltpu
from jax.experimental.pallas import tpu_sc as plsc

_LAT = 128
_N_TYPES = 9
_EPS = 1e-5

_N_PAD = 10240
_E_PAD = 122880
_NW = 32
_CHUNK = 256
_BE = 2048
_BN = 1024
_DUMMY = 10000


def _ln(h):
    m = jnp.mean(h, axis=-1, keepdims=True)
    hm = h - m
    v = jnp.mean(hm * hm, axis=-1, keepdims=True)
    return hm * lax.rsqrt(v + _EPS)


def _mm(a, b):
    return jnp.dot(a, b, preferred_element_type=jnp.float32)


def _enc_node_body(x, w0, b0, w1, b1, w2, b2, ws, wr, o_nl, o_p, o_q):
    h = jnp.maximum(_mm(x[...], w0[...]) + b0[...], 0.0)
    h = jnp.maximum(_mm(h, w1[...]) + b1[...], 0.0)
    nl = _ln(_mm(h, w2[...]) + b2[...])
    o_nl[...] = nl
    o_p[...] = _mm(nl, ws[...])
    o_q[...] = _mm(nl, wr[...])


def _enc_edge_body(x, w0, b0, w1, b1, w2, b2, o):
    h = jnp.maximum(_mm(x[...], w0[...]) + b0[...], 0.0)
    h = jnp.maximum(_mm(h, w1[...]) + b1[...], 0.0)
    o[...] = _ln(_mm(h, w2[...]) + b2[...])


def _edge_step_body(pg, qg, e, w1e, b1, w2, b2, w3, b3, o):
    h = jnp.maximum(pg[...] + qg[...] + _mm(e[...], w1e[...]) + b1[...], 0.0)
    h = jnp.maximum(_mm(h, w2[...]) + b2[...], 0.0)
    h = _mm(h, w3[...]) + b3[...]
    o[...] = e[...] + _ln(h)


def _node_step_body(nl, a0, a1, wn, wa, b1, w2, b2, w3, b3, ws, wr,
                    o_nl, o_p, o_q):
    a = a0[...] + a1[...]
    h = jnp.maximum(_mm(nl[...], wn[...]) + _mm(a, wa[...]) + b1[...], 0.0)
    h = jnp.maximum(_mm(h, w2[...]) + b2[...], 0.0)
    new = nl[...] + _ln(_mm(h, w3[...]) + b3[...])
    o_nl[...] = new
    o_p[...] = _mm(new, ws[...])
    o_q[...] = _mm(new, wr[...])


def _node_last_body(nl, a0, a1, wn, wa, b1, w2, b2, w3, b3, o_nl):
    a = a0[...] + a1[...]
    h = jnp.maximum(_mm(nl[...], wn[...]) + _mm(a, wa[...]) + b1[...], 0.0)
    h = jnp.maximum(_mm(h, w2[...]) + b2[...], 0.0)
    o_nl[...] = nl[...] + _ln(_mm(h, w3[...]) + b3[...])


def _decoder_body(nl, w0, b0, w1, b1, w2, b2, o):
    h = jnp.maximum(_mm(nl[...], w0[...]) + b0[...], 0.0)
    h = jnp.maximum(_mm(h, w1[...]) + b1[...], 0.0)
    o[...] = _mm(h, w2[...]) + b2[...]


def _row_spec(rows, cols):
    return pl.BlockSpec((rows, cols), lambda i: (i, 0))


def _w_spec(r, c):
    return pl.BlockSpec((r, c), lambda i: (0, 0))


def _tc_call(body, grid, in_specs, out_specs, out_shapes, args):
    return pl.pallas_call(
        body,
        grid=(grid,),
        in_specs=in_specs,
        out_specs=out_specs,
        out_shape=out_shapes,
    )(*args)


@functools.cache
def _sc_kernels():
    mesh = plsc.VectorSubcoreMesh(core_axis_name="c", subcore_axis_name="s")

    @functools.partial(
        pl.kernel,
        out_type=(
            jax.ShapeDtypeStruct((_E_PAD, _LAT), jnp.float32),
            jax.ShapeDtypeStruct((_E_PAD, _LAT), jnp.float32),
        ),
        mesh=mesh,
        scratch_types=[
            pltpu.VMEM((_CHUNK,), jnp.int32),
            pltpu.VMEM((_CHUNK,), jnp.int32),
            pltpu.VMEM((_CHUNK, _LAT), jnp.float32),
            pltpu.VMEM((_CHUNK, _LAT), jnp.float32),
            pltpu.SemaphoreType.DMA,
            pltpu.SemaphoreType.DMA,
        ],
    )
    def gather2(p_hbm, q_hbm, s_hbm, r_hbm, o_p, o_q,
                idx_s, idx_r, buf_p, buf_q, sem_p, sem_q):
        wid = lax.axis_index("s") * 2 + lax.axis_index("c")
        per_w = _E_PAD // _NW
        base = wid * per_w

        def body(j, carry):
            off = base + j * _CHUNK
            pltpu.sync_copy(s_hbm.at[pl.ds(off, _CHUNK)], idx_s)
            pltpu.sync_copy(r_hbm.at[pl.ds(off, _CHUNK)], idx_r)
            cp_p = pltpu.async_copy(p_hbm.at[idx_s], buf_p, sem_p)
            cp_q = pltpu.async_copy(q_hbm.at[idx_r], buf_q, sem_q)
            cp_p.wait()
            cp_q.wait()
            pltpu.sync_copy(buf_p, o_p.at[pl.ds(off, _CHUNK)])
            pltpu.sync_copy(buf_q, o_q.at[pl.ds(off, _CHUNK)])
            return carry

        lax.fori_loop(0, per_w // _CHUNK, body, 0)

    @functools.partial(
        pl.kernel,
        out_type=jax.ShapeDtypeStruct((2, _N_PAD, _LAT), jnp.float32),
        mesh=mesh,
        scratch_types=[
            pltpu.VMEM((_CHUNK,), jnp.int32),
            pltpu.VMEM((_CHUNK, _LAT), jnp.float32),
            pltpu.VMEM_SHARED((_N_PAD, _LAT), jnp.float32),
        ],
    )
    def scatter(msg_hbm, idx_hbm, zero_hbm, o_agg, idx_v, buf, agg_sh):
        cid = lax.axis_index("c")
        sid = lax.axis_index("s")

        @pl.when(sid == 0)
        def _init():
            pltpu.sync_copy(zero_hbm, agg_sh)

        plsc.subcore_barrier()

        wid = sid * 2 + cid
        per_w = _E_PAD // _NW
        base = wid * per_w

        def body(j, carry):
            off = base + j * _CHUNK
            pltpu.sync_copy(idx_hbm.at[pl.ds(off, _CHUNK)], idx_v)
            pltpu.sync_copy(msg_hbm.at[pl.ds(off, _CHUNK)], buf)
            pltpu.sync_copy(buf, agg_sh.at[idx_v], add=True)
            return carry

        lax.fori_loop(0, per_w // _CHUNK, body, 0)
        plsc.subcore_barrier()

        rows = _N_PAD // 16
        pltpu.sync_copy(agg_sh.at[pl.ds(sid * rows, rows)],
                        o_agg.at[cid, pl.ds(sid * rows, rows)])

    return gather2, scatter


def _sc_gather2(p, q, s, r):
    return _sc_kernels()[0](p, q, s, r)


def _sc_scatter(msg, idx, zeros):
    return _sc_kernels()[1](msg, idx, zeros)


def _triangles_to_edges(cells0, n_nodes):
    e = jnp.concatenate([
        cells0[:, 0:2],
        cells0[:, 1:3],
        jnp.stack([cells0[:, 2], cells0[:, 0]], axis=1),
    ], axis=0)
    lo = jnp.minimum(e[:, 0], e[:, 1])
    hi = jnp.maximum(e[:, 0], e[:, 1])
    keep = lo < hi
    sentinel = n_nodes * n_nodes
    packed = lo.astype(jnp.int64) * n_nodes + hi.astype(jnp.int64)
    packed = jnp.where(keep, packed, sentinel)
    sorted_packed = jnp.sort(packed)
    is_first = jnp.concatenate([
        jnp.ones((1,), jnp.bool_),
        sorted_packed[1:] != sorted_packed[:-1],
    ])
    mask = is_first & (sorted_packed < sentinel)
    u = jnp.where(mask, sorted_packed // n_nodes, 0)
    v = jnp.where(mask, sorted_packed % n_nodes, 0)
    senders = jnp.concatenate([u, v])
    receivers = jnp.concatenate([v, u])
    edge_mask = jnp.concatenate([mask, mask])
    return senders, receivers, edge_mask


def _normalize(x):
    mean = x.mean(axis=(0, 1), keepdims=True)
    std = jnp.sqrt(((x - mean) ** 2).mean(axis=(0, 1), keepdims=True))
    std = jnp.maximum(std, 1e-8)
    return (x - mean) / std


def _normalize_masked(x, mask):
    m = mask[None, :, None].astype(x.dtype)
    cnt = m.sum(axis=(0, 1), keepdims=True)
    mean = (x * m).sum(axis=(0, 1), keepdims=True) / cnt
    std = jnp.sqrt((((x - mean) ** 2) * m).sum(axis=(0, 1), keepdims=True) / cnt)
    std = jnp.maximum(std, 1e-8)
    return (x - mean) / std


def _pad_w(w, rows):
    return jnp.zeros((rows, w.shape[1]), w.dtype).at[: w.shape[0]].set(w)


def kernel(world_pos, prev_world_pos, node_type, mesh_pos, cells, params):
    n = world_pos.shape[1]
    f32 = jnp.float32

    senders, receivers, edge_mask = _triangles_to_edges(cells[0], n)
    e_real = senders.shape[0]
    senders = senders.astype(jnp.int32)
    receivers = receivers.astype(jnp.int32)

    velocity = world_pos - prev_world_pos
    one_hot = jax.nn.one_hot(node_type[:, :, 0], _N_TYPES, dtype=f32)
    node_features = jnp.concatenate([velocity, one_hot], axis=-1)
    rel_w = (jnp.take(world_pos, senders, axis=1)
             - jnp.take(world_pos, receivers, axis=1))
    rel_m = (jnp.take(mesh_pos, senders, axis=1)
             - jnp.take(mesh_pos, receivers, axis=1))
    edge_features = jnp.concatenate([
        rel_w,
        jnp.linalg.norm(rel_w, axis=-1, keepdims=True),
        rel_m,
        jnp.linalg.norm(rel_m, axis=-1, keepdims=True),
    ], axis=-1)
    nf = _normalize(node_features)[0]
    ef = _normalize_masked(edge_features, edge_mask)[0]

    x_node = jnp.zeros((_N_PAD, 16), f32).at[:n, :12].set(nf)
    x_edge = jnp.zeros((_E_PAD, 8), f32).at[:e_real, :7].set(ef)
    s_pad = jnp.zeros((_E_PAD,), jnp.int32).at[:e_real].set(senders)
    r_pad = jnp.zeros((_E_PAD,), jnp.int32).at[:e_real].set(receivers)
    mask_pad = jnp.zeros((_E_PAD,), jnp.bool_).at[:e_real].set(edge_mask)
    scatter_idx = jnp.where(mask_pad, r_pad, _DUMMY)
    zeros_agg = jnp.zeros((_N_PAD, _LAT), f32)

    def wb(layer, pad_rows=None):
        w = layer["W"]
        if pad_rows is not None:
            w = _pad_w(w, pad_rows)
        return w, layer["b"].reshape(1, -1)

    enc_n = params["node_encoder"]
    enc_e = params["edge_encoder"]
    dec = params["decoder"]
    blocks = params["gn_blocks"]

    step_w = []
    for blk in blocks:
        ew0 = blk["edge_mlp"][0]["W"]
        nw0 = blk["node_mlp"][0]["W"]
        step_w.append(dict(
            ws=ew0[:_LAT], wr=ew0[_LAT:2 * _LAT], we=ew0[2 * _LAT:],
            eb1=blk["edge_mlp"][0]["b"].reshape(1, -1),
            ew2=blk["edge_mlp"][1]["W"], eb2=blk["edge_mlp"][1]["b"].reshape(1, -1),
            ew3=blk["edge_mlp"][2]["W"], eb3=blk["edge_mlp"][2]["b"].reshape(1, -1),
            wn=nw0[:_LAT], wa=nw0[_LAT:],
            nb1=blk["node_mlp"][0]["b"].reshape(1, -1),
            nw2=blk["node_mlp"][1]["W"], nb2=blk["node_mlp"][1]["b"].reshape(1, -1),
            nw3=blk["node_mlp"][2]["W"], nb3=blk["node_mlp"][2]["b"].reshape(1, -1),
        ))

    n_grid = _N_PAD // _BN
    e_grid = _E_PAD // _BE
    bspec = pl.BlockSpec((1, _LAT), lambda i: (0, 0))
    nl_shape = jax.ShapeDtypeStruct((_N_PAD, _LAT), f32)
    el_shape = jax.ShapeDtypeStruct((_E_PAD, _LAT), f32)

    w0, b0 = wb(enc_n[0], 16)
    w1, b1 = wb(enc_n[1])
    w2, b2 = wb(enc_n[2])
    node_lat, p, q = _tc_call(
        _enc_node_body, n_grid,
        [_row_spec(_BN, 16), _w_spec(16, _LAT), bspec, _w_spec(_LAT, _LAT),
         bspec, _w_spec(_LAT, _LAT), bspec, _w_spec(_LAT, _LAT),
         _w_spec(_LAT, _LAT)],
        [_row_spec(_BN, _LAT)] * 3,
        (nl_shape, nl_shape, nl_shape),
        (x_node, w0, b0, w1, b1, w2, b2, step_w[0]["ws"], step_w[0]["wr"]),
    )

    w0, b0 = wb(enc_e[0], 8)
    w1, b1 = wb(enc_e[1])
    w2, b2 = wb(enc_e[2])
    edge_lat = _tc_call(
        _enc_edge_body, e_grid,
        [_row_spec(_BE, 8), _w_spec(8, _LAT), bspec, _w_spec(_LAT, _LAT),
         bspec, _w_spec(_LAT, _LAT), bspec],
        _row_spec(_BE, _LAT),
        el_shape,
        (x_edge, w0, b0, w1, b1, w2, b2),
    )

    for i, sw in enumerate(step_w):
        pg, qg = _sc_gather2(p, q, s_pad, r_pad)
        edge_lat = _tc_call(
            _edge_step_body, e_grid,
            [_row_spec(_BE, _LAT)] * 3
            + [_w_spec(_LAT, _LAT), bspec, _w_spec(_LAT, _LAT), bspec,
               _w_spec(_LAT, _LAT), bspec],
            _row_spec(_BE, _LAT),
            el_shape,
            (pg, qg, edge_lat, sw["we"], sw["eb1"], sw["ew2"], sw["eb2"],
             sw["ew3"], sw["eb3"]),
        )
        aggs = _sc_scatter(edge_lat, scatter_idx, zeros_agg)
        if i + 1 < len(step_w):
            nxt = step_w[i + 1]
            node_lat, p, q = _tc_call(
                _node_step_body, n_grid,
                [_row_spec(_BN, _LAT)] * 3
                + [_w_spec(_LAT, _LAT), _w_spec(_LAT, _LAT), bspec,
                   _w_spec(_LAT, _LAT), bspec, _w_spec(_LAT, _LAT), bspec,
                   _w_spec(_LAT, _LAT), _w_spec(_LAT, _LAT)],
                [_row_spec(_BN, _LAT)] * 3,
                (nl_shape, nl_shape, nl_shape),
                (node_lat, aggs[0], aggs[1], sw["wn"], sw["wa"], sw["nb1"],
                 sw["nw2"], sw["nb2"], sw["nw3"], sw["nb3"],
                 nxt["ws"], nxt["wr"]),
            )
        else:
            node_lat = _tc_call(
                _node_last_body, n_grid,
                [_row_spec(_BN, _LAT)] * 3
                + [_w_spec(_LAT, _LAT), _w_spec(_LAT, _LAT), bspec,
                   _w_spec(_LAT, _LAT), bspec, _w_spec(_LAT, _LAT), bspec],
                _row_spec(_BN, _LAT),
                nl_shape,
                (node_lat, aggs[0], aggs[1], sw["wn"], sw["wa"], sw["nb1"],
                 sw["nw2"], sw["nb2"], sw["nw3"], sw["nb3"]),
            )

    w0, b0 = wb(dec[0])
    w1, b1 = wb(dec[1])
    w2 = _pad_cols(dec[2]["W"])
    b2 = _pad_cols(dec[2]["b"].reshape(1, -1))
    out = _tc_call(
        _decoder_body, n_grid,
        [_row_spec(_BN, _LAT), _w_spec(_LAT, _LAT), bspec, _w_spec(_LAT, _LAT),
         bspec, _w_spec(_LAT, 8), pl.BlockSpec((1, 8), lambda i: (0, 0))],
        _row_spec(_BN, 8),
        jax.ShapeDtypeStruct((_N_PAD, 8), f32),
        (node_lat, w0, b0, w1, b1, w2, b2),
    )
    return out[:n, :3][None]


def _pad_cols(w):
    return jnp.zeros((w.shape[0], 8), w.dtype).at[:, : w.shape[1]].set(w)


# speedup vs baseline: 2.4560x; 1.1343x over previous
import functools

import jax
import jax.numpy as jnp
from jax import lax
from jax.experimental import pallas as pl
from jax.experimental.pallas import tpu as pltpu
from jax.experimental.pallas import tpu_sc as plsc

_LAT = 128
_N_TYPES = 9
_EPS = 1e-5

_N_PAD = 10240
_E_PAD = 122880
_NW = 32
_CHUNK = 256
_BE = 2048
_BN = 1024
_DUMMY = 10000


def _ln(h):
    m = jnp.mean(h, axis=-1, keepdims=True)
    hm = h - m
    v = jnp.mean(hm * hm, axis=-1, keepdims=True)
    return hm * lax.rsqrt(v + _EPS)


def _mm(a, b):
    return jnp.dot(a, b, preferred_element_type=jnp.float32)


def _enc_node_body(x, w0, b0, w1, b1, w2, b2, ws, wr, o_nl, o_p, o_q):
    h = jnp.maximum(_mm(x[...], w0[...]) + b0[...], 0.0)
    h = jnp.maximum(_mm(h, w1[...]) + b1[...], 0.0)
    nl = _ln(_mm(h, w2[...]) + b2[...])
    o_nl[...] = nl
    o_p[...] = _mm(nl, ws[...])
    o_q[...] = _mm(nl, wr[...])


def _enc_edge_body(x, w0, b0, w1, b1, w2, b2, o):
    h = jnp.maximum(_mm(x[...], w0[...]) + b0[...], 0.0)
    h = jnp.maximum(_mm(h, w1[...]) + b1[...], 0.0)
    o[...] = _ln(_mm(h, w2[...]) + b2[...])


def _edge_step_body(pg, qg, e, w1e, b1, w2, b2, w3, b3, o):
    h = jnp.maximum(pg[...] + qg[...] + _mm(e[...], w1e[...]) + b1[...], 0.0)
    h = jnp.maximum(_mm(h, w2[...]) + b2[...], 0.0)
    h = _mm(h, w3[...]) + b3[...]
    o[...] = e[...] + _ln(h)


def _node_step_body(nl, a0, a1, wn, wa, b1, w2, b2, w3, b3, ws, wr,
                    o_nl, o_p, o_q):
    a = a0[...] + a1[...]
    h = jnp.maximum(_mm(nl[...], wn[...]) + _mm(a, wa[...]) + b1[...], 0.0)
    h = jnp.maximum(_mm(h, w2[...]) + b2[...], 0.0)
    new = nl[...] + _ln(_mm(h, w3[...]) + b3[...])
    o_nl[...] = new
    o_p[...] = _mm(new, ws[...])
    o_q[...] = _mm(new, wr[...])


def _node_last_body(nl, a0, a1, wn, wa, b1, w2, b2, w3, b3, o_nl):
    a = a0[...] + a1[...]
    h = jnp.maximum(_mm(nl[...], wn[...]) + _mm(a, wa[...]) + b1[...], 0.0)
    h = jnp.maximum(_mm(h, w2[...]) + b2[...], 0.0)
    o_nl[...] = nl[...] + _ln(_mm(h, w3[...]) + b3[...])


def _decoder_body(nl, w0, b0, w1, b1, w2, b2, o):
    h = jnp.maximum(_mm(nl[...], w0[...]) + b0[...], 0.0)
    h = jnp.maximum(_mm(h, w1[...]) + b1[...], 0.0)
    o[...] = _mm(h, w2[...]) + b2[...]


def _row_spec(rows, cols):
    return pl.BlockSpec((rows, cols), lambda i: (i, 0))


def _w_spec(r, c):
    return pl.BlockSpec((r, c), lambda i: (0, 0))


def _tc_call(body, grid, in_specs, out_specs, out_shapes, args):
    return pl.pallas_call(
        body,
        grid=(grid,),
        in_specs=in_specs,
        out_specs=out_specs,
        out_shape=out_shapes,
    )(*args)


_GC = 320
_G_PER_W = _E_PAD // 16
_G_NC = _G_PER_W // _GC
_SGC = 120
_SC_PER_W = _E_PAD // _NW
_S_NC = _SC_PER_W // _SGC


@functools.cache
def _sc_kernels():
    mesh = plsc.VectorSubcoreMesh(core_axis_name="c", subcore_axis_name="s")

    @functools.partial(
        pl.kernel,
        out_type=(
            jax.ShapeDtypeStruct((_E_PAD, _LAT), jnp.float32),
            jax.ShapeDtypeStruct((_E_PAD, _LAT), jnp.float32),
        ),
        mesh=mesh,
        scratch_types=[
            pltpu.VMEM((_G_PER_W,), jnp.int32),
            pltpu.VMEM((_GC, _LAT), jnp.float32),
            pltpu.VMEM((_GC, _LAT), jnp.float32),
            pltpu.SemaphoreType.DMA,
            pltpu.SemaphoreType.DMA,
            pltpu.SemaphoreType.DMA,
            pltpu.SemaphoreType.DMA,
        ],
    )
    def gather2(p_hbm, q_hbm, s_hbm, r_hbm, o_p, o_q,
                idx_all, row0, row1, g0, g1, w0, w1):
        cid = lax.axis_index("c")
        sid = lax.axis_index("s")
        base = sid * _G_PER_W
        rows = (row0, row1)
        gsem = (g0, g1)
        wsem = (w0, w1)

        def pipeline(tab_hbm, i_hbm, out_hbm):
            pltpu.sync_copy(i_hbm.at[pl.ds(base, _G_PER_W)], idx_all)
            pltpu.async_copy(tab_hbm.at[idx_all.at[pl.ds(0, _GC)]],
                             rows[0], gsem[0])

            def body(k, carry):
                for b in (0, 1):
                    c = 2 * k + b
                    nb = 1 - b

                    @pl.when(c + 1 < _G_NC)
                    def _prefetch():
                        @pl.when(c >= 1)
                        def _free():
                            pltpu.make_async_copy(
                                rows[nb],
                                out_hbm.at[pl.ds(base, _GC)],
                                wsem[nb]).wait()
                        pltpu.async_copy(
                            tab_hbm.at[idx_all.at[pl.ds((c + 1) * _GC, _GC)]],
                            rows[nb], gsem[nb])

                    pltpu.make_async_copy(
                        tab_hbm.at[idx_all.at[pl.ds(0, _GC)]],
                        rows[b], gsem[b]).wait()
                    pltpu.async_copy(
                        rows[b], out_hbm.at[pl.ds(base + c * _GC, _GC)],
                        wsem[b])
                return carry

            lax.fori_loop(0, _G_NC // 2, body, 0)
            for b in (0, 1):
                pltpu.make_async_copy(
                    rows[b], out_hbm.at[pl.ds(base, _GC)], wsem[b]).wait()

        @pl.when(cid == 0)
        def _p():
            pipeline(p_hbm, s_hbm, o_p)

        @pl.when(cid == 1)
        def _q():
            pipeline(q_hbm, r_hbm, o_q)

    @functools.partial(
        pl.kernel,
        out_type=jax.ShapeDtypeStruct((2, _N_PAD, _LAT), jnp.float32),
        mesh=mesh,
        scratch_types=[
            pltpu.VMEM((_SGC,), jnp.int32),
            pltpu.VMEM((_SGC,), jnp.int32),
            pltpu.VMEM((_SGC, _LAT), jnp.float32),
            pltpu.VMEM((_SGC, _LAT), jnp.float32),
            pltpu.VMEM_SHARED((_N_PAD, _LAT), jnp.float32),
            pltpu.SemaphoreType.DMA,
            pltpu.SemaphoreType.DMA,
            pltpu.SemaphoreType.DMA,
            pltpu.SemaphoreType.DMA,
        ],
    )
    def scatter(msg_hbm, idx_hbm, zero_hbm, o_agg,
                idx0, idx1, buf0, buf1, agg_sh, m0, m1, i0, i1):
        cid = lax.axis_index("c")
        sid = lax.axis_index("s")

        @pl.when(sid == 0)
        def _init():
            pltpu.sync_copy(zero_hbm, agg_sh)

        wid = sid * 2 + cid
        base = wid * _SC_PER_W
        bufs = (buf0, buf1)
        idxs = (idx0, idx1)
        msem = (m0, m1)
        isem = (i0, i1)

        plsc.subcore_barrier()
        pltpu.async_copy(msg_hbm.at[pl.ds(base, _SGC)], bufs[0], msem[0])
        pltpu.async_copy(idx_hbm.at[pl.ds(base, _SGC)], idxs[0], isem[0])

        def body(k, carry):
            for b in (0, 1):
                c = 2 * k + b
                nb = 1 - b

                @pl.when(c + 1 < _S_NC)
                def _prefetch():
                    off = base + (c + 1) * _SGC
                    pltpu.async_copy(msg_hbm.at[pl.ds(off, _SGC)],
                                     bufs[nb], msem[nb])
                    pltpu.async_copy(idx_hbm.at[pl.ds(off, _SGC)],
                                     idxs[nb], isem[nb])

                pltpu.make_async_copy(
                    msg_hbm.at[pl.ds(base, _SGC)], bufs[b], msem[b]).wait()
                pltpu.make_async_copy(
                    idx_hbm.at[pl.ds(base, _SGC)], idxs[b], isem[b]).wait()
                pltpu.sync_copy(bufs[b], agg_sh.at[idxs[b]], add=True)
            return carry

        lax.fori_loop(0, _S_NC // 2, body, 0)
        plsc.subcore_barrier()

        rows = _N_PAD // 16
        pltpu.sync_copy(agg_sh.at[pl.ds(sid * rows, rows)],
                        o_agg.at[cid, pl.ds(sid * rows, rows)])

    return gather2, scatter


def _sc_gather2(p, q, s, r):
    return _sc_kernels()[0](p, q, s, r)


def _sc_scatter(msg, idx, zeros):
    return _sc_kernels()[1](msg, idx, zeros)


def _triangles_to_edges(cells0, n_nodes):
    e = jnp.concatenate([
        cells0[:, 0:2],
        cells0[:, 1:3],
        jnp.stack([cells0[:, 2], cells0[:, 0]], axis=1),
    ], axis=0)
    lo = jnp.minimum(e[:, 0], e[:, 1])
    hi = jnp.maximum(e[:, 0], e[:, 1])
    keep = lo < hi
    sentinel = n_nodes * n_nodes
    packed = lo.astype(jnp.int64) * n_nodes + hi.astype(jnp.int64)
    packed = jnp.where(keep, packed, sentinel)
    sorted_packed = jnp.sort(packed)
    is_first = jnp.concatenate([
        jnp.ones((1,), jnp.bool_),
        sorted_packed[1:] != sorted_packed[:-1],
    ])
    mask = is_first & (sorted_packed < sentinel)
    u = jnp.where(mask, sorted_packed // n_nodes, 0)
    v = jnp.where(mask, sorted_packed % n_nodes, 0)
    senders = jnp.concatenate([u, v])
    receivers = jnp.concatenate([v, u])
    edge_mask = jnp.concatenate([mask, mask])
    return senders, receivers, edge_mask


def _normalize(x):
    mean = x.mean(axis=(0, 1), keepdims=True)
    std = jnp.sqrt(((x - mean) ** 2).mean(axis=(0, 1), keepdims=True))
    std = jnp.maximum(std, 1e-8)
    return (x - mean) / std


def _normalize_masked(x, mask):
    m = mask[None, :, None].astype(x.dtype)
    cnt = m.sum(axis=(0, 1), keepdims=True)
    mean = (x * m).sum(axis=(0, 1), keepdims=True) / cnt
    std = jnp.sqrt((((x - mean) ** 2) * m).sum(axis=(0, 1), keepdims=True) / cnt)
    std = jnp.maximum(std, 1e-8)
    return (x - mean) / std


def _pad_w(w, rows):
    return jnp.zeros((rows, w.shape[1]), w.dtype).at[: w.shape[0]].set(w)


def kernel(world_pos, prev_world_pos, node_type, mesh_pos, cells, params):
    n = world_pos.shape[1]
    f32 = jnp.float32

    senders, receivers, edge_mask = _triangles_to_edges(cells[0], n)
    e_real = senders.shape[0]
    senders = senders.astype(jnp.int32)
    receivers = receivers.astype(jnp.int32)

    velocity = world_pos - prev_world_pos
    one_hot = jax.nn.one_hot(node_type[:, :, 0], _N_TYPES, dtype=f32)
    node_features = jnp.concatenate([velocity, one_hot], axis=-1)
    rel_w = (jnp.take(world_pos, senders, axis=1)
             - jnp.take(world_pos, receivers, axis=1))
    rel_m = (jnp.take(mesh_pos, senders, axis=1)
             - jnp.take(mesh_pos, receivers, axis=1))
    edge_features = jnp.concatenate([
        rel_w,
        jnp.linalg.norm(rel_w, axis=-1, keepdims=True),
        rel_m,
        jnp.linalg.norm(rel_m, axis=-1, keepdims=True),
    ], axis=-1)
    nf = _normalize(node_features)[0]
    ef = _normalize_masked(edge_features, edge_mask)[0]

    x_node = jnp.zeros((_N_PAD, 16), f32).at[:n, :12].set(nf)
    x_edge = jnp.zeros((_E_PAD, 8), f32).at[:e_real, :7].set(ef)
    s_pad = jnp.zeros((_E_PAD,), jnp.int32).at[:e_real].set(senders)
    r_pad = jnp.zeros((_E_PAD,), jnp.int32).at[:e_real].set(receivers)
    mask_pad = jnp.zeros((_E_PAD,), jnp.bool_).at[:e_real].set(edge_mask)
    scatter_idx = jnp.where(mask_pad, r_pad, _DUMMY)
    zeros_agg = jnp.zeros((_N_PAD, _LAT), f32)

    def wb(layer, pad_rows=None):
        w = layer["W"]
        if pad_rows is not None:
            w = _pad_w(w, pad_rows)
        return w, layer["b"].reshape(1, -1)

    enc_n = params["node_encoder"]
    enc_e = params["edge_encoder"]
    dec = params["decoder"]
    blocks = params["gn_blocks"]

    step_w = []
    for blk in blocks:
        ew0 = blk["edge_mlp"][0]["W"]
        nw0 = blk["node_mlp"][0]["W"]
        step_w.append(dict(
            ws=ew0[:_LAT], wr=ew0[_LAT:2 * _LAT], we=ew0[2 * _LAT:],
            eb1=blk["edge_mlp"][0]["b"].reshape(1, -1),
            ew2=blk["edge_mlp"][1]["W"], eb2=blk["edge_mlp"][1]["b"].reshape(1, -1),
            ew3=blk["edge_mlp"][2]["W"], eb3=blk["edge_mlp"][2]["b"].reshape(1, -1),
            wn=nw0[:_LAT], wa=nw0[_LAT:],
            nb1=blk["node_mlp"][0]["b"].reshape(1, -1),
            nw2=blk["node_mlp"][1]["W"], nb2=blk["node_mlp"][1]["b"].reshape(1, -1),
            nw3=blk["node_mlp"][2]["W"], nb3=blk["node_mlp"][2]["b"].reshape(1, -1),
        ))

    n_grid = _N_PAD // _BN
    e_grid = _E_PAD // _BE
    bspec = pl.BlockSpec((1, _LAT), lambda i: (0, 0))
    nl_shape = jax.ShapeDtypeStruct((_N_PAD, _LAT), f32)
    el_shape = jax.ShapeDtypeStruct((_E_PAD, _LAT), f32)

    w0, b0 = wb(enc_n[0], 16)
    w1, b1 = wb(enc_n[1])
    w2, b2 = wb(enc_n[2])
    node_lat, p, q = _tc_call(
        _enc_node_body, n_grid,
        [_row_spec(_BN, 16), _w_spec(16, _LAT), bspec, _w_spec(_LAT, _LAT),
         bspec, _w_spec(_LAT, _LAT), bspec, _w_spec(_LAT, _LAT),
         _w_spec(_LAT, _LAT)],
        [_row_spec(_BN, _LAT)] * 3,
        (nl_shape, nl_shape, nl_shape),
        (x_node, w0, b0, w1, b1, w2, b2, step_w[0]["ws"], step_w[0]["wr"]),
    )

    w0, b0 = wb(enc_e[0], 8)
    w1, b1 = wb(enc_e[1])
    w2, b2 = wb(enc_e[2])
    edge_lat = _tc_call(
        _enc_edge_body, e_grid,
        [_row_spec(_BE, 8), _w_spec(8, _LAT), bspec, _w_spec(_LAT, _LAT),
         bspec, _w_spec(_LAT, _LAT), bspec],
        _row_spec(_BE, _LAT),
        el_shape,
        (x_edge, w0, b0, w1, b1, w2, b2),
    )

    for i, sw in enumerate(step_w):
        pg, qg = _sc_gather2(p, q, s_pad, r_pad)
        edge_lat = _tc_call(
            _edge_step_body, e_grid,
            [_row_spec(_BE, _LAT)] * 3
            + [_w_spec(_LAT, _LAT), bspec, _w_spec(_LAT, _LAT), bspec,
               _w_spec(_LAT, _LAT), bspec],
            _row_spec(_BE, _LAT),
            el_shape,
            (pg, qg, edge_lat, sw["we"], sw["eb1"], sw["ew2"], sw["eb2"],
             sw["ew3"], sw["eb3"]),
        )
        aggs = _sc_scatter(edge_lat, scatter_idx, zeros_agg)
        if i + 1 < len(step_w):
            nxt = step_w[i + 1]
            node_lat, p, q = _tc_call(
                _node_step_body, n_grid,
                [_row_spec(_BN, _LAT)] * 3
                + [_w_spec(_LAT, _LAT), _w_spec(_LAT, _LAT), bspec,
                   _w_spec(_LAT, _LAT), bspec, _w_spec(_LAT, _LAT), bspec,
                   _w_spec(_LAT, _LAT), _w_spec(_LAT, _LAT)],
                [_row_spec(_BN, _LAT)] * 3,
                (nl_shape, nl_shape, nl_shape),
                (node_lat, aggs[0], aggs[1], sw["wn"], sw["wa"], sw["nb1"],
                 sw["nw2"], sw["nb2"], sw["nw3"], sw["nb3"],
                 nxt["ws"], nxt["wr"]),
            )
        else:
            node_lat = _tc_call(
                _node_last_body, n_grid,
                [_row_spec(_BN, _LAT)] * 3
                + [_w_spec(_LAT, _LAT), _w_spec(_LAT, _LAT), bspec,
                   _w_spec(_LAT, _LAT), bspec, _w_spec(_LAT, _LAT), bspec],
                _row_spec(_BN, _LAT),
                nl_shape,
                (node_lat, aggs[0], aggs[1], sw["wn"], sw["wa"], sw["nb1"],
                 sw["nw2"], sw["nb2"], sw["nw3"], sw["nb3"]),
            )

    w0, b0 = wb(dec[0])
    w1, b1 = wb(dec[1])
    w2 = _pad_cols(dec[2]["W"])
    b2 = _pad_cols(dec[2]["b"].reshape(1, -1))
    out = _tc_call(
        _decoder_body, n_grid,
        [_row_spec(_BN, _LAT), _w_spec(_LAT, _LAT), bspec, _w_spec(_LAT, _LAT),
         bspec, _w_spec(_LAT, 8), pl.BlockSpec((1, 8), lambda i: (0, 0))],
        _row_spec(_BN, 8),
        jax.ShapeDtypeStruct((_N_PAD, 8), f32),
        (node_lat, w0, b0, w1, b1, w2, b2),
    )
    return out[:n, :3][None]


def _pad_cols(w):
    return jnp.zeros((w.shape[0], 8), w.dtype).at[:, : w.shape[1]].set(w)
